# Initial kernel scaffold; baseline (speedup 1.0000x reference)
import jax
import jax.numpy as jnp
from jax.experimental import pallas as pl


def kernel(lr_xyz, hr_xyz, lr_feat, hr_feat, neigh_idx, W1, b1, W2, b2):
    raise NotImplementedError("write your pallas kernel here")


# speedup vs baseline: 6.8141x; 6.8141x over previous
import functools

import jax
import jax.numpy as jnp
from jax import lax
from jax.experimental import pallas as pl
from jax.experimental.pallas import tpu as pltpu
from jax.experimental.pallas import tpu_sc as plsc

NQ = 100000
NH = 100000
NS = 16
C = 32

NW = 32
ROWS = NQ * NS
IDXROWS_PER_TILE = 392
GROUP = 8
NSTEP = IDXROWS_PER_TILE // GROUP
IDXROWS = NW * IDXROWS_PER_TILE
PAD_ROWS = IDXROWS * 128

_PREC = lax.Precision.HIGHEST


def _prep_body(xyz_ref, feat_ref, wxyz_ref, wfeat_ref, b_ref, a_ref):
    a_ref[...] = (
        jnp.dot(xyz_ref[...], wxyz_ref[...], precision=_PREC,
                preferred_element_type=jnp.float32)
        + jnp.dot(feat_ref[...], wfeat_ref[...], precision=_PREC,
                  preferred_element_type=jnp.float32)
        + b_ref[...]
    )


_PREP_B = 5000


def _prep_call(xyz4, feat4, bdw1a, bdw1b, b1t):
    n4 = NH // 4
    grid = n4 // _PREP_B
    return pl.pallas_call(
        _prep_body,
        grid=(grid,),
        in_specs=[
            pl.BlockSpec((_PREP_B, 12), lambda i: (i, 0)),
            pl.BlockSpec((_PREP_B, 128), lambda i: (i, 0)),
            pl.BlockSpec((12, 128), lambda i: (0, 0)),
            pl.BlockSpec((128, 128), lambda i: (0, 0)),
            pl.BlockSpec((1, 128), lambda i: (0, 0)),
        ],
        out_specs=pl.BlockSpec((_PREP_B, 128), lambda i: (i, 0)),
        out_shape=jax.ShapeDtypeStruct((n4, 128), jnp.float32),
    )(xyz4, feat4, bdw1a, bdw1b, b1t)


@functools.cache
def _sc_gather_kernel():
    @functools.partial(
        pl.kernel,
        out_type=jax.ShapeDtypeStruct((PAD_ROWS, C), jnp.float32),
        mesh=plsc.VectorSubcoreMesh(
            core_axis_name="c", subcore_axis_name="s",
            num_cores=2, num_subcores=16),
        scratch_types=[
            pltpu.VMEM((GROUP, 128), jnp.int32),
            pltpu.VMEM((GROUP * 128, C), jnp.float32),
            pltpu.SemaphoreType.DMA,
        ],
        compiler_params=pltpu.CompilerParams(use_tc_tiling_on_sc=False),
    )
    def _sc_gather(table_hbm, idx_hbm, out_hbm, idx_v, rows_v, sem):
        wid = lax.axis_index("s") * 2 + lax.axis_index("c")
        base = wid * IDXROWS_PER_TILE

        def step(t, carry):
            r0 = base + t * GROUP
            pltpu.sync_copy(idx_hbm.at[pl.ds(r0, GROUP)], idx_v)
            cps = [
                pltpu.async_copy(
                    table_hbm.at[idx_v.at[j]],
                    rows_v.at[pl.ds(j * 128, 128)],
                    sem,
                )
                for j in range(GROUP)
            ]
            for cp in cps:
                cp.wait()
            pltpu.sync_copy(rows_v, out_hbm.at[pl.ds(r0 * 128, GROUP * 128)])
            return carry

        lax.fori_loop(0, NSTEP, step, 0)

    return _sc_gather


_B = 2000


def _mlp_body(g_ref, lxyz_ref, lfeat_ref, wq_ref, bdw2_ref, b2_ref, out_ref):
    q = jnp.dot(lxyz_ref[...], wq_ref[...], precision=_PREC,
                preferred_element_type=jnp.float32)
    q4 = jnp.concatenate([q, q, q, q], axis=1)
    qrep = jnp.broadcast_to(q4[:, None, :], (_B, 4, 128))
    qrep = qrep.reshape(_B * 4, 128)
    h = jnp.maximum(g_ref[...] - qrep, 0.0)
    y = jnp.dot(h, bdw2_ref[...], precision=_PREC,
                preferred_element_type=jnp.float32) + b2_ref[...]
    y = jnp.maximum(y, 0.0)
    y3 = y.reshape(_B, 4, 128)
    m1 = jnp.maximum(jnp.maximum(y3[:, 0, :], y3[:, 1, :]),
                     jnp.maximum(y3[:, 2, :], y3[:, 3, :]))
    m = jnp.maximum(jnp.maximum(m1[:, 0:32], m1[:, 32:64]),
                    jnp.maximum(m1[:, 64:96], m1[:, 96:128]))
    out_ref[...] = jnp.concatenate([lfeat_ref[...], m], axis=1)


def _mlp_call(g4, lr_xyz, lr_feat, w1a, bdw2, b2t):
    grid = NQ // _B
    gb = _B * NS // 4
    return pl.pallas_call(
        _mlp_body,
        grid=(grid,),
        in_specs=[
            pl.BlockSpec((gb, 128), lambda i: (i, 0)),
            pl.BlockSpec((_B, 3), lambda i: (i, 0)),
            pl.BlockSpec((_B, C), lambda i: (i, 0)),
            pl.BlockSpec((3, C), lambda i: (0, 0)),
            pl.BlockSpec((128, 128), lambda i: (0, 0)),
            pl.BlockSpec((1, 128), lambda i: (0, 0)),
        ],
        out_specs=pl.BlockSpec((_B, 2 * C), lambda i: (i, 0)),
        out_shape=jax.ShapeDtypeStruct((NQ, 2 * C), jnp.float32),
    )(g4, lr_xyz, lr_feat, w1a, bdw2, b2t)


def kernel(lr_xyz, hr_xyz, lr_feat, hr_feat, neigh_idx, W1, b1, W2, b2):
    w1a = W1[:3]
    w1b = W1[3:]
    eye4 = jnp.eye(4, dtype=jnp.float32)
    bdw1a = jnp.kron(eye4, w1a)
    bdw1b = jnp.kron(eye4, w1b)
    bdw2 = jnp.kron(eye4, W2)
    b1t = jnp.tile(b1, 4)[None, :]
    b2t = jnp.tile(b2, 4)[None, :]

    xyz4 = hr_xyz.reshape(NH // 4, 12)
    feat4 = hr_feat.reshape(NH // 4, 128)
    a4 = _prep_call(xyz4, feat4, bdw1a, bdw1b, b1t)
    table = a4.reshape(NH, C)

    idx = neigh_idx.astype(jnp.int32).reshape(-1)
    idx = jnp.pad(idx, (0, PAD_ROWS - ROWS))
    idx2d = idx.reshape(IDXROWS, 128)

    g = _sc_gather_kernel()(table, idx2d)
    g4 = g.reshape(PAD_ROWS // 4, 128)

    return _mlp_call(g4, lr_xyz, lr_feat, w1a, bdw2, b2t)


# speedup vs baseline: 8.6774x; 1.2734x over previous
import functools

import jax
import jax.numpy as jnp
from jax import lax
from jax.experimental import pallas as pl
from jax.experimental.pallas import tpu as pltpu
from jax.experimental.pallas import tpu_sc as plsc

NQ = 100000
NH = 100000
NS = 16
C = 32

NQP = 102400
STEPQ = 1024
NSTEP = NQP // 2 // STEPQ
NSTREAM = STEPQ // 128

_DN_T = (((0,), (0,)), ((), ()))


def _prep_body(xyzT_ref, featT_ref, w1a_ref, w1b_ref, b_ref, a_ref):
    pb = xyzT_ref.shape[1]
    a_rows = (
        lax.dot_general(xyzT_ref[...], w1a_ref[...], _DN_T,
                        preferred_element_type=jnp.float32)
        + lax.dot_general(featT_ref[...], w1b_ref[...], _DN_T,
                          preferred_element_type=jnp.float32)
    )
    a3 = a_rows.reshape(pb // 4, 4, C)
    a_ref[...] = jnp.concatenate(
        [a3[:, 0, :], a3[:, 1, :], a3[:, 2, :], a3[:, 3, :]], axis=1
    ) + b_ref[...]


_PREP_B = 6400


NH_PAD = 102400


def _prep_call(xyzT, featT, w1a, w1b, b1t):
    grid = NH_PAD // _PREP_B
    return pl.pallas_call(
        _prep_body,
        grid=(grid,),
        in_specs=[
            pl.BlockSpec((3, _PREP_B), lambda i: (0, i)),
            pl.BlockSpec((C, _PREP_B), lambda i: (0, i)),
            pl.BlockSpec((3, C), lambda i: (0, 0)),
            pl.BlockSpec((C, C), lambda i: (0, 0)),
            pl.BlockSpec((1, 128), lambda i: (0, 0)),
        ],
        out_specs=pl.BlockSpec((_PREP_B // 4, 128), lambda i: (i, 0)),
        out_shape=jax.ShapeDtypeStruct((NH_PAD // 4, 128), jnp.float32),
    )(xyzT, featT, w1a, w1b, b1t)


@functools.cache
def _sc_gather_kernel():
    @functools.partial(
        pl.kernel,
        out_type=jax.ShapeDtypeStruct((NS * NQP, C), jnp.float32),
        mesh=plsc.VectorSubcoreMesh(
            core_axis_name="c", subcore_axis_name="s",
            num_cores=2, num_subcores=16),
        scratch_types=[
            pltpu.VMEM((NSTREAM, 128), jnp.int32),
            pltpu.VMEM((NSTREAM, 128), jnp.int32),
            pltpu.VMEM((STEPQ, C), jnp.float32),
            pltpu.VMEM((STEPQ, C), jnp.float32),
            pltpu.SemaphoreType.DMA,
            pltpu.SemaphoreType.DMA,
            pltpu.SemaphoreType.DMA,
        ],
        compiler_params=pltpu.CompilerParams(use_tc_tiling_on_sc=False),
    )
    def _sc_gather(table_hbm, idxt_hbm, out_hbm,
                   iv0, iv1, rv0, rv1, sem0, sem1, sem_w):
        wid = lax.axis_index("s") * 2 + lax.axis_index("c")
        plane = wid // 2
        half = wid % 2
        rbase = half * (NQP // 2 // 128)
        qbase = plane * NQP + half * (NQP // 2)

        def load_idx(t, iv):
            r0 = rbase + t * NSTREAM
            pltpu.sync_copy(idxt_hbm.at[plane, pl.ds(r0, NSTREAM)], iv)

        def fire(iv, rv, sem):
            return [
                pltpu.async_copy(
                    table_hbm.at[iv.at[k]],
                    rv.at[pl.ds(k * 128, 128)],
                    sem,
                )
                for k in range(NSTREAM)
            ]

        def wb(t, rv):
            q0 = qbase + t * STEPQ
            return pltpu.async_copy(rv, out_hbm.at[pl.ds(q0, STEPQ)], sem_w)

        def pair(tt, carry):
            t0 = tt * 2
            t1 = t0 + 1
            load_idx(t0, iv0)
            g0 = fire(iv0, rv0, sem0)
            load_idx(t1, iv1)
            g1 = fire(iv1, rv1, sem1)
            for cp in g0:
                cp.wait()
            w0 = wb(t0, rv0)
            for cp in g1:
                cp.wait()
            w1 = wb(t1, rv1)
            w0.wait()
            w1.wait()
            return carry

        lax.fori_loop(0, NSTEP // 2, pair, 0)

    return _sc_gather


_B = 4096


def _mlp_body(g_ref, lxT_ref, lfT_ref, w1a_ref, eye_ref, bdw2_ref, b2_ref,
              out_ref):
    b4 = _B // 4
    q_rows = lax.dot_general(lxT_ref[...], w1a_ref[...], _DN_T,
                             preferred_element_type=jnp.float32)
    q3 = q_rows.reshape(b4, 4, C)
    q4 = jnp.concatenate(
        [q3[:, 0, :], q3[:, 1, :], q3[:, 2, :], q3[:, 3, :]], axis=1)
    w2 = bdw2_ref[...]
    b2 = b2_ref[...]
    acc = None
    for j in range(NS):
        h = jnp.maximum(g_ref[j] - q4, 0.0)
        y = jnp.maximum(
            jnp.dot(h, w2, preferred_element_type=jnp.float32) + b2, 0.0)
        acc = y if acc is None else jnp.maximum(acc, y)
    m3 = jnp.concatenate(
        [acc[:, None, 0:C], acc[:, None, C:2 * C],
         acc[:, None, 2 * C:3 * C], acc[:, None, 3 * C:4 * C]], axis=1)
    m_rows = m3.reshape(_B, C)
    lf_rows = lax.dot_general(lfT_ref[...], eye_ref[...], _DN_T,
                              preferred_element_type=jnp.float32)
    out_ref[...] = jnp.concatenate([lf_rows, m_rows], axis=1)


def _mlp_call(g3, lxT, lfT, w1a, eye32, bdw2, b2t):
    grid = (NQ + _B - 1) // _B
    b4 = _B // 4
    return pl.pallas_call(
        _mlp_body,
        grid=(grid,),
        in_specs=[
            pl.BlockSpec((NS, b4, 128), lambda i: (0, i, 0)),
            pl.BlockSpec((3, _B), lambda i: (0, i)),
            pl.BlockSpec((C, _B), lambda i: (0, i)),
            pl.BlockSpec((3, C), lambda i: (0, 0)),
            pl.BlockSpec((C, C), lambda i: (0, 0)),
            pl.BlockSpec((128, 128), lambda i: (0, 0)),
            pl.BlockSpec((1, 128), lambda i: (0, 0)),
        ],
        out_specs=pl.BlockSpec((_B, 2 * C), lambda i: (i, 0)),
        out_shape=jax.ShapeDtypeStruct((NQ, 2 * C), jnp.float32),
    )(g3, lxT, lfT, w1a, eye32, bdw2, b2t)


def kernel(lr_xyz, hr_xyz, lr_feat, hr_feat, neigh_idx, W1, b1, W2, b2):
    w1a = W1[:3]
    w1b = W1[3:]
    eye4 = jnp.eye(4, dtype=jnp.float32)
    eye32 = jnp.eye(C, dtype=jnp.float32)
    bdw2 = jnp.kron(eye4, W2)
    b1t = jnp.tile(b1, 4)[None, :]
    b2t = jnp.tile(b2, 4)[None, :]

    a4 = _prep_call(hr_xyz.T, hr_feat.T, w1a, w1b, b1t)
    table = a4.reshape(NH_PAD, C)

    idxt = jnp.pad(neigh_idx.astype(jnp.int32).T, ((0, 0), (0, NQP - NQ)))
    idxt = idxt.reshape(NS, NQP // 128, 128)

    g = _sc_gather_kernel()(table, idxt)
    g3 = g.reshape(NS, NQP // 4, 128)

    return _mlp_call(g3, lr_xyz.T, lr_feat.T, w1a, eye32, bdw2, b2t)


# speedup vs baseline: 9.4733x; 1.0917x over previous
import functools

import jax
import jax.numpy as jnp
from jax import lax
from jax.experimental import pallas as pl
from jax.experimental.pallas import tpu as pltpu
from jax.experimental.pallas import tpu_sc as plsc

NQ = 100000
NH = 100000
NS = 16
C = 32

NQP = 102400
STEPQ = 1024
NSTEP = NQP // 2 // STEPQ
NSTREAM = STEPQ // 128

_DN_T = (((0,), (0,)), ((), ()))


NH_PAD = 102400
_PREP_B = 6400
_PS = _PREP_B // 4


def _prep_body(xyzT_ref, featT_ref, w1a_ref, w1b_ref, b_ref, a_ref):
    a_rows = (
        lax.dot_general(xyzT_ref[...], w1a_ref[...], _DN_T,
                        preferred_element_type=jnp.float32)
        + lax.dot_general(featT_ref[...], w1b_ref[...], _DN_T,
                          preferred_element_type=jnp.float32)
    )
    a_ref[...] = jnp.concatenate(
        [a_rows[0:_PS], a_rows[_PS:2 * _PS],
         a_rows[2 * _PS:3 * _PS], a_rows[3 * _PS:4 * _PS]], axis=1
    ) + b_ref[...]


def _prep_call(xyzT, featT, w1a, w1b, b1t):
    grid = NH_PAD // _PREP_B
    return pl.pallas_call(
        _prep_body,
        grid=(grid,),
        in_specs=[
            pl.BlockSpec((3, _PREP_B), lambda i: (0, i)),
            pl.BlockSpec((C, _PREP_B), lambda i: (0, i)),
            pl.BlockSpec((3, C), lambda i: (0, 0)),
            pl.BlockSpec((C, C), lambda i: (0, 0)),
            pl.BlockSpec((1, 128), lambda i: (0, 0)),
        ],
        out_specs=pl.BlockSpec((_PREP_B // 4, 128), lambda i: (i, 0)),
        out_shape=jax.ShapeDtypeStruct((NH_PAD // 4, 128), jnp.float32),
    )(xyzT, featT, w1a, w1b, b1t)


@functools.cache
def _sc_gather_kernel():
    @functools.partial(
        pl.kernel,
        out_type=jax.ShapeDtypeStruct((NS * NQP, C), jnp.float32),
        mesh=plsc.VectorSubcoreMesh(
            core_axis_name="c", subcore_axis_name="s",
            num_cores=2, num_subcores=16),
        scratch_types=[
            pltpu.VMEM((NSTREAM, 128), jnp.int32),
            pltpu.VMEM((NSTREAM, 128), jnp.int32),
            pltpu.VMEM((STEPQ, C), jnp.float32),
            pltpu.VMEM((STEPQ, C), jnp.float32),
            pltpu.SemaphoreType.DMA,
            pltpu.SemaphoreType.DMA,
            pltpu.SemaphoreType.DMA,
        ],
        compiler_params=pltpu.CompilerParams(use_tc_tiling_on_sc=False),
    )
    def _sc_gather(table_hbm, idxt_hbm, out_hbm,
                   iv0, iv1, rv0, rv1, sem0, sem1, sem_w):
        wid = lax.axis_index("s") * 2 + lax.axis_index("c")
        plane = wid // 2
        half = wid % 2
        rbase = half * (NQP // 2 // 128)
        qbase = plane * NQP + half * (NQP // 2)

        def load_idx(t, iv):
            r0 = rbase + t * NSTREAM
            pltpu.sync_copy(idxt_hbm.at[plane, pl.ds(r0, NSTREAM)], iv)

        def fire(iv, rv, sem):
            return [
                pltpu.async_copy(
                    table_hbm.at[iv.at[k]],
                    rv.at[pl.ds(k * 128, 128)],
                    sem,
                )
                for k in range(NSTREAM)
            ]

        def wb(t, rv):
            q0 = qbase + t * STEPQ
            return pltpu.async_copy(rv, out_hbm.at[pl.ds(q0, STEPQ)], sem_w)

        def pair(tt, carry):
            t0 = tt * 2
            t1 = t0 + 1
            load_idx(t0, iv0)
            g0 = fire(iv0, rv0, sem0)
            load_idx(t1, iv1)
            g1 = fire(iv1, rv1, sem1)
            for cp in g0:
                cp.wait()
            w0 = wb(t0, rv0)
            for cp in g1:
                cp.wait()
            w1 = wb(t1, rv1)
            w0.wait()
            w1.wait()
            return carry

        lax.fori_loop(0, NSTEP // 2, pair, 0)

    return _sc_gather


_B = 4096


def _mlp_body(g_ref, lxT_ref, lfT_ref, w1a_ref, eye_ref, bdw2_ref, b2_ref,
              out_ref):
    b4 = _B // 4
    q_rows = lax.dot_general(lxT_ref[...], w1a_ref[...], _DN_T,
                             preferred_element_type=jnp.float32)
    qid = pl.program_id(0) * _B + lax.broadcasted_iota(jnp.int32, (_B, C), 0)
    q_rows = jnp.where(qid < NQ, q_rows, 0.0)
    q4 = jnp.concatenate(
        [q_rows[0:b4], q_rows[b4:2 * b4],
         q_rows[2 * b4:3 * b4], q_rows[3 * b4:4 * b4]], axis=1)
    w2 = bdw2_ref[...]
    b2 = b2_ref[...]
    acc = None
    for j in range(NS):
        h = jnp.maximum(g_ref[j] - q4, 0.0)
        y = jnp.maximum(
            jnp.dot(h, w2, preferred_element_type=jnp.float32) + b2, 0.0)
        acc = y if acc is None else jnp.maximum(acc, y)
    m_rows = jnp.concatenate(
        [acc[:, 0:C], acc[:, C:2 * C],
         acc[:, 2 * C:3 * C], acc[:, 3 * C:4 * C]], axis=0)
    lf_rows = lax.dot_general(lfT_ref[...], eye_ref[...], _DN_T,
                              preferred_element_type=jnp.float32)
    out_ref[...] = jnp.concatenate([lf_rows, m_rows], axis=1)


def _mlp_call(g3, lxT, lfT, w1a, eye32, bdw2, b2t):
    grid = (NQ + _B - 1) // _B
    b4 = _B // 4
    return pl.pallas_call(
        _mlp_body,
        grid=(grid,),
        in_specs=[
            pl.BlockSpec((NS, b4, 128), lambda i: (0, i, 0)),
            pl.BlockSpec((3, _B), lambda i: (0, i)),
            pl.BlockSpec((C, _B), lambda i: (0, i)),
            pl.BlockSpec((3, C), lambda i: (0, 0)),
            pl.BlockSpec((C, C), lambda i: (0, 0)),
            pl.BlockSpec((128, 128), lambda i: (0, 0)),
            pl.BlockSpec((1, 128), lambda i: (0, 0)),
        ],
        out_specs=pl.BlockSpec((_B, 2 * C), lambda i: (i, 0)),
        out_shape=jax.ShapeDtypeStruct((NQ, 2 * C), jnp.float32),
    )(g3, lxT, lfT, w1a, eye32, bdw2, b2t)


def kernel(lr_xyz, hr_xyz, lr_feat, hr_feat, neigh_idx, W1, b1, W2, b2):
    w1a = W1[:3]
    w1b = W1[3:]
    eye4 = jnp.eye(4, dtype=jnp.float32)
    eye32 = jnp.eye(C, dtype=jnp.float32)
    bdw2 = jnp.kron(eye4, W2)
    b1t = jnp.tile(b1, 4)[None, :]
    b2t = jnp.tile(b2, 4)[None, :]

    a4 = _prep_call(hr_xyz.T, hr_feat.T, w1a, w1b, b1t)
    table = a4.reshape(NH_PAD, C)

    v = neigh_idx.astype(jnp.int32).T
    loc = v % _PREP_B
    vrow = (v - loc) + 4 * (loc % _PS) + loc // _PS

    idxt = jnp.pad(vrow, ((0, 0), (0, NQP - NQ)))
    idxt = idxt.reshape(NS, NQP // _B, 4, _B // 4).transpose(0, 1, 3, 2)
    idxt = idxt.reshape(NS, NQP // 128, 128)

    g = _sc_gather_kernel()(table, idxt)
    g3 = g.reshape(NS, NQP // 4, 128)

    return _mlp_call(g3, lr_xyz.T, lr_feat.T, w1a, eye32, bdw2, b2t)
